# baseline (speedup 1.0000x reference)
import functools

import jax
import jax.numpy as jnp
from jax import lax
from jax.experimental import pallas as pl
from jax.experimental.pallas import tpu as pltpu
from jax.experimental.pallas import tpu_sc as plsc

VOCAB = 30522
H = 768
NC, NS = 2, 16
NW = NC * NS
B, S = 4, 2048
T = B * S
CH = 64
NBUF = 2
EPS = 1e-12

NSPLIT = 1
TP = T // NSPLIT
BP = B // NSPLIT
PER_W = TP // NW
NCHUNK = PER_W // CH

BS = 2048
NSB = S // BS


def _sc_gather_body(ids_hbm, word_hbm, out_hbm, idx_all, *scratch):
    bufs = scratch[0:NBUF]
    sem_g = scratch[NBUF:2 * NBUF]
    sem_o = scratch[2 * NBUF:3 * NBUF]

    wid = lax.axis_index("s") * NC + lax.axis_index("c")
    base0 = wid * PER_W

    pltpu.sync_copy(ids_hbm.at[pl.ds(base0, PER_W)], idx_all)

    cps_g = [None] * NCHUNK
    cps_o = [None] * NCHUNK
    for c in range(min(NBUF, NCHUNK)):
        cps_g[c] = pltpu.async_copy(
            word_hbm.at[idx_all.at[pl.ds(c * CH, CH)]], bufs[c], sem_g[c])
    for c in range(NCHUNK):
        b = c % NBUF
        cps_g[c].wait()
        cps_o[c] = pltpu.async_copy(
            bufs[b], out_hbm.at[pl.ds(base0 + c * CH, CH)], sem_o[b])
        n = c + NBUF
        if n < NCHUNK:
            cps_o[c].wait()
            cps_g[n] = pltpu.async_copy(
                word_hbm.at[idx_all.at[pl.ds(n * CH, CH)]], bufs[b], sem_g[b])
    for c in range(max(0, NCHUNK - NBUF), NCHUNK):
        cps_o[c].wait()


def _sc_gather(ids, word_emb):
    mesh = plsc.VectorSubcoreMesh(core_axis_name="c", subcore_axis_name="s",
                                  num_cores=NC, num_subcores=NS)
    scratch = ([pltpu.VMEM((PER_W,), jnp.int32)]
               + [pltpu.VMEM((CH, H), jnp.float32) for _ in range(NBUF)]
               + [pltpu.SemaphoreType.DMA for _ in range(2 * NBUF)])
    f = pl.kernel(
        _sc_gather_body,
        out_type=jax.ShapeDtypeStruct((TP, H), jnp.float32),
        mesh=mesh,
        scratch_types=scratch,
    )
    return f(ids, word_emb)


def _tc_ln_body(x_ref, pos_ref, tti_ref, tok_ref, gam_ref, bet_ref, o_ref):
    x = x_ref[...]
    t = tti_ref[...].astype(jnp.float32)
    tok0 = tok_ref[0:1, :]
    tokrow = tok0 + t * (tok_ref[1:2, :] - tok0)
    x = x + pos_ref[...] + tokrow
    m = jnp.mean(x, axis=1, keepdims=True)
    xc = x - m
    var = jnp.mean(xc * xc, axis=1, keepdims=True)
    inv = lax.rsqrt(var + EPS)
    o_ref[...] = xc * inv * gam_ref[...] + bet_ref[...]


def _tc_ln(gathered, pos_emb, ttf, tok_type_emb, ln_gamma, ln_beta):
    grid = (NSB, BP)
    return pl.pallas_call(
        _tc_ln_body,
        grid=grid,
        in_specs=[
            pl.BlockSpec((BS, H), lambda s, b: (b * NSB + s, 0)),
            pl.BlockSpec((BS, H), lambda s, b: (s, 0)),
            pl.BlockSpec((BS, 1), lambda s, b: (b * NSB + s, 0)),
            pl.BlockSpec((2, H), lambda s, b: (0, 0)),
            pl.BlockSpec((H,), lambda s, b: (0,)),
            pl.BlockSpec((H,), lambda s, b: (0,)),
        ],
        out_specs=pl.BlockSpec((BS, H), lambda s, b: (b * NSB + s, 0)),
        out_shape=jax.ShapeDtypeStruct((TP, H), jnp.float32),
    )(gathered, pos_emb, ttf, tok_type_emb, ln_gamma, ln_beta)


@jax.jit
def _run(ids, ttf, word_emb, pos_emb, tok_type_emb, ln_gamma, ln_beta):
    parts = []
    gs = [_sc_gather(ids[p * TP:(p + 1) * TP], word_emb)
          for p in range(NSPLIT)]
    for p in range(NSPLIT):
        parts.append(_tc_ln(gs[p], pos_emb, ttf[p * TP:(p + 1) * TP],
                            tok_type_emb, ln_gamma, ln_beta))
    return jnp.concatenate(parts, axis=0)


def kernel(input_ids, token_type_ids, word_emb, pos_emb, tok_type_emb,
           ln_gamma, ln_beta):
    ids = input_ids.reshape(T).astype(jnp.int32)
    ttf = token_type_ids.reshape(T, 1).astype(jnp.float32)
    out = _run(ids, ttf, word_emb, pos_emb, tok_type_emb, ln_gamma, ln_beta)
    return out.reshape(B, S, H)


# speedup vs baseline: 1.0146x; 1.0146x over previous
import functools

import jax
import jax.numpy as jnp
from jax import lax
from jax.experimental import pallas as pl
from jax.experimental.pallas import tpu as pltpu
from jax.experimental.pallas import tpu_sc as plsc

VOCAB = 30522
H = 768
NC, NS = 2, 16
NW = NC * NS
B, S = 4, 2048
T = B * S
CH = 32
NBUF = 5
EPS = 1e-12

NSPLIT = 1
TP = T // NSPLIT
BP = B // NSPLIT
PER_W = TP // NW
NCHUNK = PER_W // CH

BS = 1024
NSB = S // BS


def _sc_gather_body(ids_hbm, word_hbm, out_hbm, idx_all, *scratch):
    bufs = scratch[0:NBUF]
    sem_g = scratch[NBUF:2 * NBUF]
    sem_o = scratch[2 * NBUF:3 * NBUF]

    wid = lax.axis_index("s") * NC + lax.axis_index("c")
    base0 = wid * PER_W

    pltpu.sync_copy(ids_hbm.at[pl.ds(base0, PER_W)], idx_all)

    cps_g = [None] * NCHUNK
    cps_o = [None] * NCHUNK
    for c in range(min(NBUF, NCHUNK)):
        cps_g[c] = pltpu.async_copy(
            word_hbm.at[idx_all.at[pl.ds(c * CH, CH)]], bufs[c], sem_g[c])
    for c in range(NCHUNK):
        b = c % NBUF
        cps_g[c].wait()
        cps_o[c] = pltpu.async_copy(
            bufs[b], out_hbm.at[pl.ds(base0 + c * CH, CH)], sem_o[b])
        n = c + NBUF
        if n < NCHUNK:
            cps_o[c].wait()
            cps_g[n] = pltpu.async_copy(
                word_hbm.at[idx_all.at[pl.ds(n * CH, CH)]], bufs[b], sem_g[b])
    for c in range(max(0, NCHUNK - NBUF), NCHUNK):
        cps_o[c].wait()


def _sc_gather(ids, word_emb):
    mesh = plsc.VectorSubcoreMesh(core_axis_name="c", subcore_axis_name="s",
                                  num_cores=NC, num_subcores=NS)
    scratch = ([pltpu.VMEM((PER_W,), jnp.int32)]
               + [pltpu.VMEM((CH, H), jnp.float32) for _ in range(NBUF)]
               + [pltpu.SemaphoreType.DMA for _ in range(2 * NBUF)])
    f = pl.kernel(
        _sc_gather_body,
        out_type=jax.ShapeDtypeStruct((TP, H), jnp.float32),
        mesh=mesh,
        scratch_types=scratch,
    )
    return f(ids, word_emb)


def _tc_ln_body(x_ref, pos_ref, tti_ref, tok_ref, gam_ref, bet_ref, o_ref):
    x = x_ref[...]
    t = tti_ref[...].astype(jnp.float32)
    tok0 = tok_ref[0:1, :]
    tokrow = tok0 + t * (tok_ref[1:2, :] - tok0)
    x = x + pos_ref[...] + tokrow
    m = jnp.mean(x, axis=1, keepdims=True)
    xc = x - m
    var = jnp.mean(xc * xc, axis=1, keepdims=True)
    inv = lax.rsqrt(var + EPS)
    o_ref[...] = xc * inv * gam_ref[...] + bet_ref[...]


def _tc_ln(gathered, pos_emb, ttf, tok_type_emb, ln_gamma, ln_beta):
    grid = (NSB, BP)
    return pl.pallas_call(
        _tc_ln_body,
        grid=grid,
        in_specs=[
            pl.BlockSpec((BS, H), lambda s, b: (b * NSB + s, 0)),
            pl.BlockSpec((BS, H), lambda s, b: (s, 0)),
            pl.BlockSpec((BS, 1), lambda s, b: (b * NSB + s, 0)),
            pl.BlockSpec((2, H), lambda s, b: (0, 0)),
            pl.BlockSpec((H,), lambda s, b: (0,)),
            pl.BlockSpec((H,), lambda s, b: (0,)),
        ],
        out_specs=pl.BlockSpec((BS, H), lambda s, b: (b * NSB + s, 0)),
        out_shape=jax.ShapeDtypeStruct((TP, H), jnp.float32),
    )(gathered, pos_emb, ttf, tok_type_emb, ln_gamma, ln_beta)


@jax.jit
def _run(ids, ttf, word_emb, pos_emb, tok_type_emb, ln_gamma, ln_beta):
    parts = []
    gs = [_sc_gather(ids[p * TP:(p + 1) * TP], word_emb)
          for p in range(NSPLIT)]
    for p in range(NSPLIT):
        parts.append(_tc_ln(gs[p], pos_emb, ttf[p * TP:(p + 1) * TP],
                            tok_type_emb, ln_gamma, ln_beta))
    return jnp.concatenate(parts, axis=0)


def kernel(input_ids, token_type_ids, word_emb, pos_emb, tok_type_emb,
           ln_gamma, ln_beta):
    ids = input_ids.reshape(T).astype(jnp.int32)
    ttf = token_type_ids.reshape(T, 1).astype(jnp.float32)
    out = _run(ids, ttf, word_emb, pos_emb, tok_type_emb, ln_gamma, ln_beta)
    return out.reshape(B, S, H)


# speedup vs baseline: 1.0205x; 1.0058x over previous
import functools

import jax
import jax.numpy as jnp
from jax import lax
from jax.experimental import pallas as pl
from jax.experimental.pallas import tpu as pltpu
from jax.experimental.pallas import tpu_sc as plsc

VOCAB = 30522
H = 768
NC, NS = 2, 16
NW = NC * NS
B, S = 4, 2048
T = B * S
CH = 32
NBUF = 5
EPS = 1e-12

NSPLIT = 1
TP = T // NSPLIT
BP = B // NSPLIT
PER_W = TP // NW
NCHUNK = PER_W // CH

BS = 2048
NSB = S // BS


def _sc_gather_body(ids_hbm, word_hbm, out_hbm, idx_all, *scratch):
    bufs = scratch[0:NBUF]
    sem_g = scratch[NBUF:2 * NBUF]
    sem_o = scratch[2 * NBUF:3 * NBUF]

    wid = lax.axis_index("s") * NC + lax.axis_index("c")
    base0 = wid * PER_W

    pltpu.sync_copy(ids_hbm.at[pl.ds(base0, PER_W)], idx_all)

    cps_g = [None] * NCHUNK
    cps_o = [None] * NCHUNK
    for c in range(min(NBUF, NCHUNK)):
        cps_g[c] = pltpu.async_copy(
            word_hbm.at[idx_all.at[pl.ds(c * CH, CH)]], bufs[c], sem_g[c])
    for c in range(NCHUNK):
        b = c % NBUF
        cps_g[c].wait()
        cps_o[c] = pltpu.async_copy(
            bufs[b], out_hbm.at[pl.ds(base0 + c * CH, CH)], sem_o[b])
        n = c + NBUF
        if n < NCHUNK:
            cps_o[c].wait()
            cps_g[n] = pltpu.async_copy(
                word_hbm.at[idx_all.at[pl.ds(n * CH, CH)]], bufs[b], sem_g[b])
    for c in range(max(0, NCHUNK - NBUF), NCHUNK):
        cps_o[c].wait()


def _sc_gather(ids, word_emb):
    mesh = plsc.VectorSubcoreMesh(core_axis_name="c", subcore_axis_name="s",
                                  num_cores=NC, num_subcores=NS)
    scratch = ([pltpu.VMEM((PER_W,), jnp.int32)]
               + [pltpu.VMEM((CH, H), jnp.float32) for _ in range(NBUF)]
               + [pltpu.SemaphoreType.DMA for _ in range(2 * NBUF)])
    f = pl.kernel(
        _sc_gather_body,
        out_type=jax.ShapeDtypeStruct((TP, H), jnp.float32),
        mesh=mesh,
        scratch_types=scratch,
    )
    return f(ids, word_emb)


def _tc_ln_body(x_ref, pos_ref, tti_ref, tok_ref, gam_ref, bet_ref, o_ref):
    x = x_ref[...]
    t = tti_ref[...].astype(jnp.float32)
    tok0 = tok_ref[0:1, :]
    tokrow = tok0 + t * (tok_ref[1:2, :] - tok0)
    x = x + pos_ref[...] + tokrow
    m = jnp.mean(x, axis=1, keepdims=True)
    xc = x - m
    var = jnp.mean(xc * xc, axis=1, keepdims=True)
    inv = lax.rsqrt(var + EPS)
    o_ref[...] = xc * inv * gam_ref[...] + bet_ref[...]


def _tc_ln(gathered, pos_emb, ttf, tok_type_emb, ln_gamma, ln_beta):
    grid = (NSB, BP)
    return pl.pallas_call(
        _tc_ln_body,
        grid=grid,
        in_specs=[
            pl.BlockSpec((BS, H), lambda s, b: (b * NSB + s, 0)),
            pl.BlockSpec((BS, H), lambda s, b: (s, 0)),
            pl.BlockSpec((BS, 1), lambda s, b: (b * NSB + s, 0)),
            pl.BlockSpec((2, H), lambda s, b: (0, 0)),
            pl.BlockSpec((H,), lambda s, b: (0,)),
            pl.BlockSpec((H,), lambda s, b: (0,)),
        ],
        out_specs=pl.BlockSpec((BS, H), lambda s, b: (b * NSB + s, 0)),
        out_shape=jax.ShapeDtypeStruct((TP, H), jnp.float32),
    )(gathered, pos_emb, ttf, tok_type_emb, ln_gamma, ln_beta)


@jax.jit
def _run(ids, ttf, word_emb, pos_emb, tok_type_emb, ln_gamma, ln_beta):
    parts = []
    gs = [_sc_gather(ids[p * TP:(p + 1) * TP], word_emb)
          for p in range(NSPLIT)]
    for p in range(NSPLIT):
        parts.append(_tc_ln(gs[p], pos_emb, ttf[p * TP:(p + 1) * TP],
                            tok_type_emb, ln_gamma, ln_beta))
    out = parts[0] if NSPLIT == 1 else jnp.concatenate(parts, axis=0)
    return out.reshape(B, S, H)


def kernel(input_ids, token_type_ids, word_emb, pos_emb, tok_type_emb,
           ln_gamma, ln_beta):
    ids = input_ids.reshape(T).astype(jnp.int32)
    ttf = token_type_ids.reshape(T, 1).astype(jnp.float32)
    return _run(ids, ttf, word_emb, pos_emb, tok_type_emb, ln_gamma, ln_beta)


# speedup vs baseline: 1.0240x; 1.0034x over previous
import functools

import jax
import jax.numpy as jnp
from jax import lax
from jax.experimental import pallas as pl
from jax.experimental.pallas import tpu as pltpu
from jax.experimental.pallas import tpu_sc as plsc

VOCAB = 30522
H = 768
NC, NS = 2, 16
NW = NC * NS
B, S = 4, 2048
T = B * S
CH = 32
NBUF = 5
EPS = 1e-12

NSPLIT = 1
TP = T // NSPLIT
BP = B // NSPLIT
PER_W = TP // NW
NCHUNK = PER_W // CH

BS = 2048
NSB = S // BS


def _sc_gather_body(ids_hbm, word_hbm, out_hbm, idx_all, *scratch):
    bufs = scratch[0:NBUF]
    sem_g = scratch[NBUF:2 * NBUF]
    sem_o = scratch[2 * NBUF:3 * NBUF]

    wid = lax.axis_index("s") * NC + lax.axis_index("c")
    base0 = wid * PER_W

    pltpu.sync_copy(ids_hbm.at[pl.ds(base0, PER_W)], idx_all)

    cps_g = [None] * NCHUNK
    cps_o = [None] * NCHUNK
    for c in range(min(NBUF, NCHUNK)):
        cps_g[c] = pltpu.async_copy(
            word_hbm.at[idx_all.at[pl.ds(c * CH, CH)]], bufs[c], sem_g[c])
    for c in range(NCHUNK):
        b = c % NBUF
        cps_g[c].wait()
        cps_o[c] = pltpu.async_copy(
            bufs[b], out_hbm.at[pl.ds(base0 + c * CH, CH)], sem_o[b])
        n = c + NBUF
        if n < NCHUNK:
            cps_o[c].wait()
            cps_g[n] = pltpu.async_copy(
                word_hbm.at[idx_all.at[pl.ds(n * CH, CH)]], bufs[b], sem_g[b])
    for c in range(max(0, NCHUNK - NBUF), NCHUNK):
        cps_o[c].wait()


def _sc_gather(ids, word_emb):
    mesh = plsc.VectorSubcoreMesh(core_axis_name="c", subcore_axis_name="s",
                                  num_cores=NC, num_subcores=NS)
    scratch = ([pltpu.VMEM((PER_W,), jnp.int32)]
               + [pltpu.VMEM((CH, H), jnp.float32) for _ in range(NBUF)]
               + [pltpu.SemaphoreType.DMA for _ in range(2 * NBUF)])
    f = pl.kernel(
        _sc_gather_body,
        out_type=jax.ShapeDtypeStruct((TP, H), jnp.float32),
        mesh=mesh,
        scratch_types=scratch,
    )
    return f(ids, word_emb)


def _tc_ln_body(x_ref, pos_ref, tti_ref, tok_ref, gam_ref, bet_ref, o_ref):
    x = x_ref[...]
    t = tti_ref[...].astype(jnp.float32)
    tok0 = tok_ref[0:1, :]
    tokrow = tok0 + t * (tok_ref[1:2, :] - tok0)
    x = x + pos_ref[...] + tokrow
    m = jnp.mean(x, axis=1, keepdims=True)
    xc = x - m
    var = jnp.mean(xc * xc, axis=1, keepdims=True)
    inv = lax.rsqrt(var + EPS)
    o_ref[...] = xc * inv * gam_ref[...] + bet_ref[...]


def _tc_ln(gathered, pos_emb, ttf, tok_type_emb, ln_gamma, ln_beta):
    grid = (NSB, BP)
    return pl.pallas_call(
        _tc_ln_body,
        grid=grid,
        in_specs=[
            pl.BlockSpec((BS, H), lambda s, b: (b * NSB + s, 0)),
            pl.BlockSpec((BS, H), lambda s, b: (s, 0)),
            pl.BlockSpec((BS, 1), lambda s, b: (b * NSB + s, 0)),
            pl.BlockSpec((2, H), lambda s, b: (0, 0)),
            pl.BlockSpec((H,), lambda s, b: (0,)),
            pl.BlockSpec((H,), lambda s, b: (0,)),
        ],
        out_specs=pl.BlockSpec((BS, H), lambda s, b: (b * NSB + s, 0)),
        out_shape=jax.ShapeDtypeStruct((TP, H), jnp.float32),
    )(gathered, pos_emb, ttf, tok_type_emb, ln_gamma, ln_beta)


@jax.jit
def _run(input_ids, token_type_ids, word_emb, pos_emb, tok_type_emb,
         ln_gamma, ln_beta):
    ids = input_ids.reshape(T).astype(jnp.int32)
    ttf = token_type_ids.reshape(T, 1).astype(jnp.float32)
    parts = []
    gs = [_sc_gather(ids[p * TP:(p + 1) * TP], word_emb)
          for p in range(NSPLIT)]
    for p in range(NSPLIT):
        parts.append(_tc_ln(gs[p], pos_emb, ttf[p * TP:(p + 1) * TP],
                            tok_type_emb, ln_gamma, ln_beta))
    out = parts[0] if NSPLIT == 1 else jnp.concatenate(parts, axis=0)
    return out.reshape(B, S, H)


def kernel(input_ids, token_type_ids, word_emb, pos_emb, tok_type_emb,
           ln_gamma, ln_beta):
    return _run(input_ids, token_type_ids, word_emb, pos_emb, tok_type_emb,
                ln_gamma, ln_beta)
